# baseline (speedup 1.0000x reference)
import functools

import jax
import jax.numpy as jnp
from jax.experimental import pallas as pl
from jax.experimental.pallas import tpu as pltpu

NUM_EXPERTS = 64
HIDDEN = 4096
TOKENS = 8192
SINKHORN_ITERS = 30
TILE = 512
CHUNK = 512


def _router_kernel(x0_ref, x1_ref, w_ref, b_ref, logits_ref, aff_ref, idx_ref,
                   costT_ref):
    i = pl.program_id(0)
    w = w_ref[...]
    b = b_ref[...]
    half = TILE // 2
    for j, x_ref in enumerate((x0_ref, x1_ref)):
        x = x_ref[...].reshape(half, HIDDEN)
        logits = jax.lax.dot_general(
            x, w, (((1,), (0,)), ((), ())),
            preferred_element_type=jnp.float32,
        ) + b
        logits_ref[pl.ds(j * half, half), :] = logits
        aff_ref[pl.ds(j * half, half), :] = jax.nn.sigmoid(logits)
        costT_ref[:, pl.ds(i * TILE + j * half, half)] = jnp.exp(logits).T

    @pl.when(i == pl.num_programs(0) - 1)
    def _sinkhorn():
        eps = 1e-8
        n_ch = TOKENS // CHUNK

        def body(_, d1):
            def chunk_step(c, acc):
                blk = costT_ref[:, pl.ds(c * CHUNK, CHUNK)]
                s = jnp.sum(blk * d1, axis=0, keepdims=True)
                d0 = (1.0 / TOKENS) / (s + eps)
                return acc + jnp.sum(blk * d0, axis=1, keepdims=True)

            v = jax.lax.fori_loop(0, n_ch, chunk_step,
                                  jnp.zeros((NUM_EXPERTS, 1), jnp.float32))
            return (1.0 / NUM_EXPERTS) / (v + eps)

        d1 = jax.lax.fori_loop(0, SINKHORN_ITERS, body,
                               jnp.ones((NUM_EXPERTS, 1), jnp.float32))
        m = costT_ref[...] * d1
        maxv = jnp.max(m, axis=0, keepdims=True)
        eidx = jax.lax.broadcasted_iota(jnp.int32, (NUM_EXPERTS, TOKENS), 0)
        idx_ref[...] = jnp.min(
            jnp.where(m == maxv, eidx, NUM_EXPERTS), axis=0, keepdims=True)


@functools.partial(jax.jit, static_argnames=())
def kernel(hidden_states, W, b):
    n_tiles = TOKENS // TILE
    logits, aff, idx = pl.pallas_call(
        _router_kernel,
        grid=(n_tiles,),
        in_specs=[
            pl.BlockSpec((TILE // 8, 4, HIDDEN), lambda i: (2 * i, 0, 0)),
            pl.BlockSpec((TILE // 8, 4, HIDDEN), lambda i: (2 * i + 1, 0, 0)),
            pl.BlockSpec((HIDDEN, NUM_EXPERTS), lambda i: (0, 0)),
            pl.BlockSpec((1, NUM_EXPERTS), lambda i: (0, 0)),
        ],
        out_specs=[
            pl.BlockSpec((TILE, NUM_EXPERTS), lambda i: (i, 0)),
            pl.BlockSpec((TILE, NUM_EXPERTS), lambda i: (i, 0)),
            pl.BlockSpec((1, TOKENS), lambda i: (0, 0)),
        ],
        out_shape=[
            jax.ShapeDtypeStruct((TOKENS, NUM_EXPERTS), jnp.float32),
            jax.ShapeDtypeStruct((TOKENS, NUM_EXPERTS), jnp.float32),
            jax.ShapeDtypeStruct((1, TOKENS), jnp.int32),
        ],
        scratch_shapes=[pltpu.VMEM((NUM_EXPERTS, TOKENS), jnp.float32)],
    )(hidden_states, hidden_states, W, b.reshape(1, NUM_EXPERTS))

    return (logits, aff, idx.reshape(TOKENS, 1))


# speedup vs baseline: 1.5760x; 1.5760x over previous
import functools

import jax
import jax.numpy as jnp
from jax.experimental import pallas as pl
from jax.experimental.pallas import tpu as pltpu

NUM_EXPERTS = 64
HIDDEN = 4096
TOKENS = 8192
SINKHORN_ITERS = 30
TILE = 512
CHUNK = 512


def _router_kernel(x0_ref, x1_ref, w_ref, b_ref, logits_ref, aff_ref, idx_ref,
                   costT_ref):
    i = pl.program_id(0)
    w = w_ref[...]
    b = b_ref[...]
    half = TILE // 2
    for j, x_ref in enumerate((x0_ref, x1_ref)):
        x = x_ref[...].reshape(half, HIDDEN)
        logits = jax.lax.dot_general(
            x, w, (((1,), (0,)), ((), ())),
            preferred_element_type=jnp.float32,
        ) + b
        logits_ref[pl.ds(j * half, half), :] = logits
        aff_ref[pl.ds(j * half, half), :] = jax.nn.sigmoid(logits)
        costT_ref[:, pl.ds(i * TILE + j * half, half)] = jnp.exp(logits).T

    @pl.when(i == pl.num_programs(0) - 1)
    def _sinkhorn():
        eps = 1e-8
        n_ch = TOKENS // CHUNK

        def body(_, d1):
            v = jnp.zeros((NUM_EXPERTS, 1), jnp.float32)
            for c in range(n_ch):
                blk = costT_ref[:, c * CHUNK:(c + 1) * CHUNK]
                s = jnp.sum(blk * d1, axis=0, keepdims=True)
                d0 = (1.0 / TOKENS) / (s + eps)
                v = v + jnp.sum(blk * d0, axis=1, keepdims=True)
            return (1.0 / NUM_EXPERTS) / (v + eps)

        d1 = jax.lax.fori_loop(0, SINKHORN_ITERS, body,
                               jnp.ones((NUM_EXPERTS, 1), jnp.float32))
        m = costT_ref[...] * d1
        maxv = jnp.max(m, axis=0, keepdims=True)
        eidx = jax.lax.broadcasted_iota(jnp.int32, (NUM_EXPERTS, TOKENS), 0)
        idx_ref[...] = jnp.min(
            jnp.where(m == maxv, eidx, NUM_EXPERTS), axis=0, keepdims=True)


@functools.partial(jax.jit, static_argnames=())
def kernel(hidden_states, W, b):
    n_tiles = TOKENS // TILE
    logits, aff, idx = pl.pallas_call(
        _router_kernel,
        grid=(n_tiles,),
        in_specs=[
            pl.BlockSpec((TILE // 8, 4, HIDDEN), lambda i: (2 * i, 0, 0)),
            pl.BlockSpec((TILE // 8, 4, HIDDEN), lambda i: (2 * i + 1, 0, 0)),
            pl.BlockSpec((HIDDEN, NUM_EXPERTS), lambda i: (0, 0)),
            pl.BlockSpec((1, NUM_EXPERTS), lambda i: (0, 0)),
        ],
        out_specs=[
            pl.BlockSpec((TILE, NUM_EXPERTS), lambda i: (i, 0)),
            pl.BlockSpec((TILE, NUM_EXPERTS), lambda i: (i, 0)),
            pl.BlockSpec((1, TOKENS), lambda i: (0, 0)),
        ],
        out_shape=[
            jax.ShapeDtypeStruct((TOKENS, NUM_EXPERTS), jnp.float32),
            jax.ShapeDtypeStruct((TOKENS, NUM_EXPERTS), jnp.float32),
            jax.ShapeDtypeStruct((1, TOKENS), jnp.int32),
        ],
        scratch_shapes=[pltpu.VMEM((NUM_EXPERTS, TOKENS), jnp.float32)],
    )(hidden_states, hidden_states, W, b.reshape(1, NUM_EXPERTS))

    return (logits, aff, idx.reshape(TOKENS, 1))
